# baseline (speedup 1.0000x reference)
import jax
import jax.numpy as jnp
from jax.experimental import pallas as pl
from jax.experimental.pallas import tpu as pltpu

_KB = 4000
_CH = 80
_BIG = 1e9


def _nearest_kernel(xt_ref, x2_ref, c_ref, c2_ref, out_ref, best_d_ref,
                    best_i_ref):
    b = pl.program_id(0)
    nb = pl.num_programs(0)
    q = xt_ref.shape[1]
    c = c_ref[...]
    xt = xt_ref[...]
    x2 = x2_ref[...]
    c2 = c2_ref[...]

    acc_v = jnp.full((8, q), jnp.inf, jnp.float32)
    acc_j = jnp.zeros((8, q), jnp.float32)
    j = 0
    for cidx in range(_KB // _CH):
        lo = cidx * _CH
        s = jax.lax.dot_general(
            c[lo:lo + _CH, :], xt, (((1,), (0,)), ((), ())),
            precision=jax.lax.Precision.DEFAULT,
            preferred_element_type=jnp.float32)
        d2 = (x2 + c2[lo:lo + _CH, :]) - 2.0 * s
        d2 = jnp.maximum(d2, 0.0)
        dist = jnp.sqrt(d2)
        for r in range(_CH // 8):
            v = dist[r * 8:(r + 1) * 8, :]
            pred = v < acc_v
            acc_v = jnp.where(pred, v, acc_v)
            acc_j = jnp.where(pred, jnp.float32(j), acc_j)
            j += 1

    rows = acc_j * 8.0 + jax.lax.broadcasted_iota(
        jnp.int32, (8, 1), 0).astype(jnp.float32)
    lmin = jnp.min(acc_v, axis=0, keepdims=True)
    larg = jnp.min(jnp.where(acc_v == lmin, rows, _BIG), axis=0,
                   keepdims=True)

    @pl.when(b == 0)
    def _init():
        best_d_ref[...] = lmin
        best_i_ref[...] = larg

    @pl.when(b != 0)
    def _merge():
        bd = best_d_ref[...]
        bi = best_i_ref[...]
        upd = lmin < bd
        off = (b * _KB).astype(jnp.float32)
        best_d_ref[...] = jnp.where(upd, lmin, bd)
        best_i_ref[...] = jnp.where(upd, larg + off, bi)

    @pl.when(b == nb - 1)
    def _emit():
        out_ref[...] = best_i_ref[...].astype(jnp.int32)


def kernel(x, centroids):
    q, d = x.shape
    k = centroids.shape[0]
    xt = x.T
    x2 = jnp.sum(x * x, axis=1)[None, :]
    c2 = jnp.sum(centroids * centroids, axis=1)[:, None]
    rem = (-k) % _KB
    if rem:
        centroids = jnp.pad(centroids, ((0, rem), (0, 0)))
        c2 = jnp.pad(c2, ((0, rem), (0, 0)), constant_values=jnp.inf)
    nb = (k + rem) // _KB
    out = pl.pallas_call(
        _nearest_kernel,
        grid=(nb,),
        in_specs=[
            pl.BlockSpec((d, q), lambda i: (0, 0)),
            pl.BlockSpec((1, q), lambda i: (0, 0)),
            pl.BlockSpec((_KB, d), lambda i: (i, 0)),
            pl.BlockSpec((_KB, 1), lambda i: (i, 0)),
        ],
        out_specs=pl.BlockSpec((1, q), lambda i: (0, 0)),
        out_shape=jax.ShapeDtypeStruct((1, q), jnp.int32),
        scratch_shapes=[
            pltpu.VMEM((1, q), jnp.float32),
            pltpu.VMEM((1, q), jnp.float32),
        ],
    )(xt, x2, centroids, c2)
    return out.reshape(q)


# speedup vs baseline: 1.1853x; 1.1853x over previous
import jax
import jax.numpy as jnp
from jax.experimental import pallas as pl
from jax.experimental.pallas import tpu as pltpu

_KB = 4000
_CH = 80
_BIG = 1e9


def _nearest_kernel(xt_ref, x2_ref, c_ref, out_ref, best_d_ref,
                    best_i_ref):
    b = pl.program_id(0)
    nb = pl.num_programs(0)
    q = xt_ref.shape[1]
    c = c_ref[...]
    xt = xt_ref[...]
    x2 = x2_ref[...]
    c2 = jnp.sum(c * c, axis=1, keepdims=True)

    acc_v = jnp.full((8, q), jnp.inf, jnp.float32)
    acc_j = jnp.zeros((8, q), jnp.float32)
    j = 0
    for cidx in range(_KB // _CH):
        lo = cidx * _CH
        s = jax.lax.dot_general(
            c[lo:lo + _CH, :], xt, (((1,), (0,)), ((), ())),
            precision=jax.lax.Precision.DEFAULT,
            preferred_element_type=jnp.float32)
        d2 = (x2 + c2[lo:lo + _CH, :]) - 2.0 * s
        d2 = jnp.maximum(d2, 0.0)
        dist = jnp.sqrt(d2)
        for r in range(_CH // 8):
            v = dist[r * 8:(r + 1) * 8, :]
            pred = v < acc_v
            acc_v = jnp.where(pred, v, acc_v)
            acc_j = jnp.where(pred, jnp.float32(j), acc_j)
            j += 1

    rows = acc_j * 8.0 + jax.lax.broadcasted_iota(
        jnp.int32, (8, 1), 0).astype(jnp.float32)
    lmin = jnp.min(acc_v, axis=0, keepdims=True)
    larg = jnp.min(jnp.where(acc_v == lmin, rows, _BIG), axis=0,
                   keepdims=True)

    @pl.when(b == 0)
    def _init():
        best_d_ref[...] = lmin
        best_i_ref[...] = larg

    @pl.when(b != 0)
    def _merge():
        bd = best_d_ref[...]
        bi = best_i_ref[...]
        upd = lmin < bd
        off = (b * _KB).astype(jnp.float32)
        best_d_ref[...] = jnp.where(upd, lmin, bd)
        best_i_ref[...] = jnp.where(upd, larg + off, bi)

    @pl.when(b == nb - 1)
    def _emit():
        out_ref[...] = best_i_ref[...].astype(jnp.int32)


def kernel(x, centroids):
    q, d = x.shape
    k = centroids.shape[0]
    xt = x.T
    x2 = jnp.sum(x * x, axis=1)[None, :]
    rem = (-k) % _KB
    if rem:
        centroids = jnp.pad(centroids, ((0, rem), (0, 0)),
                            constant_values=1e19)
    nb = (k + rem) // _KB
    out = pl.pallas_call(
        _nearest_kernel,
        grid=(nb,),
        in_specs=[
            pl.BlockSpec((d, q), lambda i: (0, 0)),
            pl.BlockSpec((1, q), lambda i: (0, 0)),
            pl.BlockSpec((_KB, d), lambda i: (i, 0)),
        ],
        out_specs=pl.BlockSpec((1, q), lambda i: (0, 0)),
        out_shape=jax.ShapeDtypeStruct((1, q), jnp.int32),
        scratch_shapes=[
            pltpu.VMEM((1, q), jnp.float32),
            pltpu.VMEM((1, q), jnp.float32),
        ],
    )(xt, x2, centroids)
    return out.reshape(q)


# speedup vs baseline: 1.4834x; 1.2515x over previous
import jax
import jax.numpy as jnp
from jax.experimental import pallas as pl
from jax.experimental.pallas import tpu as pltpu

_KB = 4000
_CH = 80
_BIG = 1e9


def _bits(x):
    return jax.lax.bitcast_convert_type(x, jnp.int32)


def _f32(x):
    return jax.lax.bitcast_convert_type(x, jnp.float32)


def _nearest_kernel(xt_ref, x2_ref, c_ref, c2r_ref, out_ref, d2_ref,
                    best_d_ref, best_i_ref):
    b = pl.program_id(0)
    nb = pl.num_programs(0)
    q = xt_ref.shape[1]
    c = c_ref[...]
    xt = xt_ref[...]
    x2 = x2_ref[...]
    c2 = jnp.transpose(c2r_ref[0], (1, 0))

    acc_v = jnp.full((8, q), jnp.inf, jnp.float32)
    for cidx in range(_KB // _CH):
        lo = cidx * _CH
        s = jax.lax.dot_general(
            c[lo:lo + _CH, :], xt, (((1,), (0,)), ((), ())),
            precision=jax.lax.Precision.DEFAULT,
            preferred_element_type=jnp.float32)
        d2 = (x2 + c2[lo:lo + _CH, :]) - 2.0 * s
        d2 = jnp.maximum(d2, 0.0)
        d2_ref[lo:lo + _CH, :] = d2
        for r in range(_CH // 8):
            acc_v = jnp.minimum(acc_v, d2[r * 8:(r + 1) * 8, :])
    d2min = jnp.min(acc_v, axis=0, keepdims=True)

    u = jnp.sqrt(d2min)
    pb = u * u
    hi = d2min
    for koff in (-3, -2, -1, 0, 1, 2, 3, 4, 5, 6):
        cand = _f32(_bits(pb) + koff)
        ok = (jnp.sqrt(cand) == u) & (cand > hi)
        hi = jnp.where(ok, cand, hi)
    hi = jnp.where(jnp.isnan(hi), d2min, hi)

    acc_j = jnp.full((8, q), _BIG, jnp.float32)
    j = 0
    for cidx in range(_KB // _CH):
        lo = cidx * _CH
        blk = d2_ref[lo:lo + _CH, :]
        for r in range(_CH // 8):
            v = blk[r * 8:(r + 1) * 8, :]
            acc_j = jnp.minimum(acc_j,
                                jnp.where(v <= hi, jnp.float32(j), _BIG))
            j += 1
    rows = acc_j * 8.0 + jax.lax.broadcasted_iota(
        jnp.int32, (8, 1), 0).astype(jnp.float32)
    larg = jnp.min(rows, axis=0, keepdims=True)

    @pl.when(b == 0)
    def _init():
        best_d_ref[...] = u
        best_i_ref[...] = larg

    @pl.when(b != 0)
    def _merge():
        bd = best_d_ref[...]
        bi = best_i_ref[...]
        upd = u < bd
        off = (b * _KB).astype(jnp.float32)
        best_d_ref[...] = jnp.where(upd, u, bd)
        best_i_ref[...] = jnp.where(upd, larg + off, bi)

    @pl.when(b == nb - 1)
    def _emit():
        out_ref[...] = best_i_ref[...].astype(jnp.int32)


def _pallas_labels(x, centroids):
    q, d = x.shape
    k = centroids.shape[0]
    xt = x.T
    x2 = jnp.sum(x * x, axis=1)[None, :]
    nb = k // _KB
    c2row = jnp.sum(centroids * centroids,
                    axis=1).reshape(nb, 1, _KB)
    out = pl.pallas_call(
        _nearest_kernel,
        grid=(nb,),
        in_specs=[
            pl.BlockSpec((d, q), lambda i: (0, 0)),
            pl.BlockSpec((1, q), lambda i: (0, 0)),
            pl.BlockSpec((_KB, d), lambda i: (i, 0)),
            pl.BlockSpec((1, 1, _KB), lambda i: (i, 0, 0)),
        ],
        out_specs=pl.BlockSpec((1, q), lambda i: (0, 0)),
        out_shape=jax.ShapeDtypeStruct((1, q), jnp.int32),
        scratch_shapes=[
            pltpu.VMEM((_KB, q), jnp.float32),
            pltpu.VMEM((1, q), jnp.float32),
            pltpu.VMEM((1, q), jnp.float32),
        ],
    )(xt, x2, centroids, c2row)
    return out.reshape(q)


def kernel(x, centroids):
    return _pallas_labels(x, centroids)


# speedup vs baseline: 1.5728x; 1.0602x over previous
import jax
import jax.numpy as jnp
from jax.experimental import pallas as pl
from jax.experimental.pallas import tpu as pltpu

_KB = 4000
_CH = 80
_BIG = 1e9


def _bits(x):
    return jax.lax.bitcast_convert_type(x, jnp.int32)


def _f32(x):
    return jax.lax.bitcast_convert_type(x, jnp.float32)


def _nearest_kernel(xt_ref, x2_ref, c_ref, c2r_ref, out_ref, d2_ref,
                    uhi_ref, best_d_ref, best_i_ref):
    b = pl.program_id(0)
    nb = pl.num_programs(0) - 1
    q = xt_ref.shape[1]

    @pl.when(b < nb)
    def _pass1():
        c = c_ref[...]
        xt = xt_ref[...]
        x2 = x2_ref[...]
        c2 = jnp.transpose(c2r_ref[0], (1, 0))
        p = jax.lax.rem(b, 2)
        acc_v = jnp.full((8, q), jnp.inf, jnp.float32)
        for cidx in range(_KB // _CH):
            lo = cidx * _CH
            s = jax.lax.dot_general(
                c[lo:lo + _CH, :], xt, (((1,), (0,)), ((), ())),
                precision=jax.lax.Precision.DEFAULT,
                preferred_element_type=jnp.float32)
            d2 = (x2 + c2[lo:lo + _CH, :]) - 2.0 * s
            d2 = jnp.maximum(d2, 0.0)
            d2_ref[p, lo:lo + _CH, :] = d2
            for r in range(_CH // 8):
                acc_v = jnp.minimum(acc_v, d2[r * 8:(r + 1) * 8, :])
        d2min = jnp.min(acc_v, axis=0, keepdims=True)

        u = jnp.sqrt(d2min)
        pb = u * u
        hi = d2min
        for koff in (-3, -2, -1, 0, 1, 2, 3, 4, 5, 6):
            cand = _f32(_bits(pb) + koff)
            ok = (jnp.sqrt(cand) == u) & (cand > hi)
            hi = jnp.where(ok, cand, hi)
        hi = jnp.where(jnp.isnan(hi), d2min, hi)
        uhi_ref[p, 0:1, :] = u
        uhi_ref[p, 1:2, :] = hi

    @pl.when(b > 0)
    def _pass2_prev():
        p = jax.lax.rem(b + 1, 2)
        u = uhi_ref[p, 0:1, :]
        hi = uhi_ref[p, 1:2, :]
        acc_j = jnp.full((8, q), _BIG, jnp.float32)
        j = 0
        for cidx in range(_KB // _CH):
            lo = cidx * _CH
            blk = d2_ref[p, lo:lo + _CH, :]
            for r in range(_CH // 8):
                v = blk[r * 8:(r + 1) * 8, :]
                acc_j = jnp.minimum(
                    acc_j, jnp.where(v <= hi, jnp.float32(j), _BIG))
                j += 1
        rows = acc_j * 8.0 + jax.lax.broadcasted_iota(
            jnp.int32, (8, 1), 0).astype(jnp.float32)
        larg = jnp.min(rows, axis=0, keepdims=True)

        @pl.when(b == 1)
        def _init():
            best_d_ref[...] = u
            best_i_ref[...] = larg

        @pl.when(b > 1)
        def _merge():
            bd = best_d_ref[...]
            bi = best_i_ref[...]
            upd = u < bd
            off = ((b - 1) * _KB).astype(jnp.float32)
            best_d_ref[...] = jnp.where(upd, u, bd)
            best_i_ref[...] = jnp.where(upd, larg + off, bi)

    @pl.when(b == nb)
    def _emit():
        out_ref[...] = best_i_ref[...].astype(jnp.int32)


def kernel(x, centroids):
    q, d = x.shape
    k = centroids.shape[0]
    xt = x.T
    x2 = jnp.sum(x * x, axis=1)[None, :]
    nb = k // _KB
    c2row = jnp.sum(centroids * centroids,
                    axis=1).reshape(nb, 1, _KB)

    def _cblk(i):
        return (jnp.minimum(i, nb - 1), 0)

    def _c2blk(i):
        return (jnp.minimum(i, nb - 1), 0, 0)

    out = pl.pallas_call(
        _nearest_kernel,
        grid=(nb + 1,),
        in_specs=[
            pl.BlockSpec((d, q), lambda i: (0, 0)),
            pl.BlockSpec((1, q), lambda i: (0, 0)),
            pl.BlockSpec((_KB, d), _cblk),
            pl.BlockSpec((1, 1, _KB), _c2blk),
        ],
        out_specs=pl.BlockSpec((1, q), lambda i: (0, 0)),
        out_shape=jax.ShapeDtypeStruct((1, q), jnp.int32),
        scratch_shapes=[
            pltpu.VMEM((2, _KB, q), jnp.float32),
            pltpu.VMEM((2, 2, q), jnp.float32),
            pltpu.VMEM((1, q), jnp.float32),
            pltpu.VMEM((1, q), jnp.float32),
        ],
    )(xt, x2, centroids, c2row)
    return out.reshape(q)
